# baseline (speedup 1.0000x reference)
import functools

import jax
import jax.numpy as jnp
from jax import lax
from jax.experimental import pallas as pl
from jax.experimental.pallas import tpu as pltpu
from jax.experimental.pallas import tpu_sc as plsc

_CHUNK_ROWS = 512
_NBUF = 3


def kernel(x, pos_table):
    seq_len = x.shape[1]
    emb_dim = pos_table.shape[1]

    info = plsc.get_sparse_core_info()
    num_cores = info.num_cores
    assert seq_len % (num_cores * _CHUNK_ROWS) == 0
    rows_per_worker = seq_len // num_cores
    nchunks = rows_per_worker // _CHUNK_ROWS

    mesh = plsc.ScalarSubcoreMesh(axis_name="c", num_cores=num_cores)

    @functools.partial(
        pl.kernel,
        mesh=mesh,
        out_type=jax.ShapeDtypeStruct((seq_len, emb_dim), jnp.float32),
        scratch_types=(
            [pltpu.VMEM_SHARED((_CHUNK_ROWS, emb_dim), jnp.float32)
             for _ in range(_NBUF)]
            + [pltpu.SemaphoreType.DMA, pltpu.SemaphoreType.DMA]
        ),
    )
    def copy_rows(table_hbm, out_hbm, *rest):
        bufs, (lsem, ssem) = rest[:_NBUF], rest[_NBUF:]
        base = lax.axis_index("c") * rows_per_worker

        def load(i):
            return pltpu.make_async_copy(
                table_hbm.at[pl.ds(base + i * _CHUNK_ROWS, _CHUNK_ROWS)],
                bufs[i % _NBUF], lsem)

        def store(i):
            return pltpu.make_async_copy(
                bufs[i % _NBUF],
                out_hbm.at[pl.ds(base + i * _CHUNK_ROWS, _CHUNK_ROWS)], ssem)

        la = _NBUF - 1
        store_waited = [False] * nchunks
        for j in range(min(la, nchunks)):
            load(j).start()
        for i in range(nchunks):
            load(i).wait()
            store(i).start()
            if i + la < nchunks:
                if i - 1 >= 0:
                    store(i - 1).wait()
                    store_waited[i - 1] = True
                load(i + la).start()
        for i in range(nchunks):
            if not store_waited[i]:
                store(i).wait()

    return copy_rows(pos_table)[None]


# speedup vs baseline: 1.0155x; 1.0155x over previous
import functools

import jax
import jax.numpy as jnp
from jax import lax
from jax.experimental import pallas as pl
from jax.experimental.pallas import tpu as pltpu
from jax.experimental.pallas import tpu_sc as plsc

_CHUNK_ROWS = 16
_NBUF = 6
_LA = 3


def kernel(x, pos_table):
    seq_len = x.shape[1]
    emb_dim = pos_table.shape[1]

    info = plsc.get_sparse_core_info()
    num_cores, num_subcores = info.num_cores, info.num_subcores
    num_workers = num_cores * num_subcores
    assert seq_len % (num_workers * _CHUNK_ROWS) == 0
    rows_per_worker = seq_len // num_workers
    nchunks = rows_per_worker // _CHUNK_ROWS

    mesh = plsc.VectorSubcoreMesh(core_axis_name="c", subcore_axis_name="s")

    @functools.partial(
        pl.kernel,
        mesh=mesh,
        out_type=jax.ShapeDtypeStruct((seq_len, emb_dim), jnp.float32),
        scratch_types=(
            [pltpu.VMEM((_CHUNK_ROWS, emb_dim), jnp.float32) for _ in range(_NBUF)]
            + [pltpu.SemaphoreType.DMA, pltpu.SemaphoreType.DMA]
        ),
    )
    def copy_rows(table_hbm, out_hbm, *rest):
        bufs, (lsem, ssem) = rest[:_NBUF], rest[_NBUF:]
        wid = lax.axis_index("s") * num_cores + lax.axis_index("c")
        base = wid * rows_per_worker

        def load(i):
            return pltpu.make_async_copy(
                table_hbm.at[pl.ds(base + i * _CHUNK_ROWS, _CHUNK_ROWS)],
                bufs[i % _NBUF], lsem)

        def store(i):
            return pltpu.make_async_copy(
                bufs[i % _NBUF],
                out_hbm.at[pl.ds(base + i * _CHUNK_ROWS, _CHUNK_ROWS)], ssem)

        store_waited = [False] * nchunks
        for j in range(min(_LA, nchunks)):
            load(j).start()
        for i in range(nchunks):
            load(i).wait()
            store(i).start()
            if i + _LA < nchunks:
                f = i + _LA - _NBUF
                if f >= 0:
                    store(f).wait()
                    store_waited[f] = True
                load(i + _LA).start()
        for i in range(nchunks):
            if not store_waited[i]:
                store(i).wait()

    return copy_rows(pos_table)[None]
